# Initial kernel scaffold; baseline (speedup 1.0000x reference)
import jax
import jax.numpy as jnp
from jax.experimental import pallas as pl


def kernel(features, edges, eps, W1, b1, W_edge, b_edge):
    raise NotImplementedError("write your pallas kernel here")


# speedup vs baseline: 14.5694x; 14.5694x over previous
import functools

import jax
import jax.numpy as jnp
from jax import lax
from jax.experimental import pallas as pl
from jax.experimental.pallas import tpu as pltpu
from jax.experimental.pallas import tpu_sc as plsc

EOS = 1e-10
NNODES = 10000
NEDGES = 320000
IN_DIM = 128
HID = 128
ALPHA = 1.0
TEMP = 1.0
BIAS = 0.0001

LANE = 128
ROWS = NEDGES // LANE
NTILES = 32
TROWS = 80
RPAD = NTILES * TROWS

_f32 = jnp.float32
_i32 = jnp.int32


def _tc_prep_body(f_ref, w1_ref, b1_ref, we_ref, be_ref, eps_ref, q_ref, g_ref):
    h = jnp.dot(f_ref[...], w1_ref[...], preferred_element_type=_f32)
    h = jnp.maximum(h + b1_ref[...], 0.0)
    w2 = 0.5 * (we_ref[:HID, :] + we_ref[HID:, :])
    q_ref[...] = jnp.dot(h, w2, preferred_element_type=_f32)
    e = eps_ref[...]
    eb = (BIAS - (1.0 - BIAS)) * e + (1.0 - BIAS)
    g_ref[...] = jnp.log(eb) - jnp.log(1.0 - eb) + be_ref[0, 0]


_tc_prep = pl.pallas_call(
    _tc_prep_body,
    out_shape=(
        jax.ShapeDtypeStruct((NNODES, 1), _f32),
        jax.ShapeDtypeStruct((RPAD, LANE), _f32),
    ),
)


def _tc_norm_body(dlp0_ref, dlp1_ref, dhp0_ref, dhp1_ref,
                  ilp_ref, ihp_ref, tlp_ref):
    dl = dlp0_ref[...] + dlp1_ref[...] + (1.0 + EOS)
    dh = dhp0_ref[...] + dhp1_ref[...] + (1.0 + EOS)
    ilp_ref[...] = lax.rsqrt(dl)
    ihp_ref[...] = lax.rsqrt(dh)
    tlp_ref[...] = (1.0 + EOS) / dl


_tc_norm = pl.pallas_call(
    _tc_norm_body,
    out_shape=(
        jax.ShapeDtypeStruct((1, NNODES), _f32),
        jax.ShapeDtypeStruct((1, NNODES), _f32),
        jax.ShapeDtypeStruct((1, NNODES), _f32),
    ),
)


_MESH = plsc.VectorSubcoreMesh(core_axis_name="c", subcore_axis_name="s",
                               num_cores=2, num_subcores=16)


@functools.partial(
    pl.kernel,
    out_type=(
        jax.ShapeDtypeStruct((RPAD, LANE), _f32),
        jax.ShapeDtypeStruct((RPAD, LANE), _f32),
        jax.ShapeDtypeStruct((NNODES,), _f32),
        jax.ShapeDtypeStruct((NNODES,), _f32),
        jax.ShapeDtypeStruct((NNODES,), _f32),
        jax.ShapeDtypeStruct((NNODES,), _f32),
    ),
    mesh=_MESH,
    compiler_params=pltpu.CompilerParams(needs_layout_passes=False),
    scratch_types=[
        pltpu.VMEM((NNODES,), _f32),
        pltpu.VMEM((NNODES,), _f32),
        pltpu.VMEM((TROWS, LANE), _i32),
        pltpu.VMEM((TROWS, LANE), _i32),
        pltpu.VMEM((TROWS, LANE), _f32),
        pltpu.VMEM((TROWS, LANE), _f32),
        pltpu.VMEM((TROWS, LANE), _f32),
        pltpu.VMEM((TROWS, LANE), _f32),
        pltpu.VMEM((TROWS, LANE), _f32),
        pltpu.VMEM_SHARED((NNODES,), _f32),
        pltpu.VMEM_SHARED((NNODES,), _f32),
    ],
)
def _sc_weights(q_hbm, src_hbm, dst_hbm, g_hbm,
                wlp_hbm, whp_hbm, dlp0_hbm, dlp1_hbm, dhp0_hbm, dhp1_hbm,
                q_v, z_v, src_v, dst_v, g_v, wlp_v, whp_v, wlpe_v, whpe_v,
                sh_lp, sh_hp):
    c = lax.axis_index("c")
    s = lax.axis_index("s")
    wid = s * 2 + c
    start = wid * TROWS
    nrows = jnp.minimum(TROWS, ROWS - start)

    pltpu.sync_copy(q_hbm, q_v)
    pltpu.sync_copy(src_hbm.at[pl.ds(start, TROWS)], src_v)
    pltpu.sync_copy(dst_hbm.at[pl.ds(start, TROWS)], dst_v)
    pltpu.sync_copy(g_hbm.at[pl.ds(start, TROWS)], g_v)

    @pl.when(s == 0)
    def _init_shared():
        def zbody(i, carry):
            z_v[pl.ds(i * 16, 16)] = jnp.zeros((16,), _f32)
            return carry
        lax.fori_loop(0, NNODES // 16, zbody, 0)
        pltpu.sync_copy(z_v, sh_lp)
        pltpu.sync_copy(z_v, sh_hp)

    plsc.subcore_barrier()

    def row_body(j, carry):
        for k in range(LANE // 16):
            sl = pl.ds(k * 16, 16)
            si = src_v[j, sl]
            di = dst_v[j, sl]
            qs = plsc.load_gather(q_v, [si])
            qd = plsc.load_gather(q_v, [di])
            x = (g_v[j, sl] + qs + qd) / TEMP
            w = 1.0 / (1.0 + jnp.exp(-x))
            wlp_v[j, sl] = w
            whp_v[j, sl] = 1.0 - w
            wlpe_v[j, sl] = w + EOS
            whpe_v[j, sl] = (1.0 - w) + EOS
        pltpu.sync_copy(wlpe_v.at[j], sh_lp.at[dst_v.at[j]], add=True)
        pltpu.sync_copy(whpe_v.at[j], sh_hp.at[dst_v.at[j]], add=True)
        return carry

    lax.fori_loop(0, nrows, row_body, 0)

    pltpu.sync_copy(wlp_v, wlp_hbm.at[pl.ds(start, TROWS)])
    pltpu.sync_copy(whp_v, whp_hbm.at[pl.ds(start, TROWS)])

    plsc.subcore_barrier()

    @pl.when((s == 0) & (c == 0))
    def _writeback_c0():
        pltpu.sync_copy(sh_lp, dlp0_hbm)
        pltpu.sync_copy(sh_hp, dhp0_hbm)

    @pl.when((s == 0) & (c == 1))
    def _writeback_c1():
        pltpu.sync_copy(sh_lp, dlp1_hbm)
        pltpu.sync_copy(sh_hp, dhp1_hbm)


@functools.partial(
    pl.kernel,
    out_type=(
        jax.ShapeDtypeStruct((RPAD, LANE), _f32),
        jax.ShapeDtypeStruct((RPAD, LANE), _f32),
    ),
    mesh=_MESH,
    compiler_params=pltpu.CompilerParams(needs_layout_passes=False),
    scratch_types=[
        pltpu.VMEM((NNODES,), _f32),
        pltpu.VMEM((NNODES,), _f32),
        pltpu.VMEM((TROWS, LANE), _i32),
        pltpu.VMEM((TROWS, LANE), _i32),
        pltpu.VMEM((TROWS, LANE), _f32),
        pltpu.VMEM((TROWS, LANE), _f32),
        pltpu.VMEM((TROWS, LANE), _f32),
        pltpu.VMEM((TROWS, LANE), _f32),
        pltpu.VMEM((TROWS, LANE), _i32),
        pltpu.VMEM((LANE,), _f32),
    ],
)
def _sc_norm_scatter(ilp_hbm, ihp_hbm, src_hbm, dst_hbm, wlp_hbm, whp_hbm,
                     adj_hbm, olp_hbm, ohp_hbm,
                     ilp_v, ihp_v, src_v, dst_v, wlp_v, whp_v,
                     olp_v, ohp_v, fidx_v, ones_v):
    c = lax.axis_index("c")
    s = lax.axis_index("s")
    wid = s * 2 + c
    start = wid * TROWS
    nrows = jnp.minimum(TROWS, ROWS - start)

    pltpu.sync_copy(ilp_hbm, ilp_v)
    pltpu.sync_copy(ihp_hbm, ihp_v)
    pltpu.sync_copy(src_hbm.at[pl.ds(start, TROWS)], src_v)
    pltpu.sync_copy(dst_hbm.at[pl.ds(start, TROWS)], dst_v)
    pltpu.sync_copy(wlp_hbm.at[pl.ds(start, TROWS)], wlp_v)
    pltpu.sync_copy(whp_hbm.at[pl.ds(start, TROWS)], whp_v)
    for k in range(LANE // 16):
        ones_v[pl.ds(k * 16, 16)] = jnp.full((16,), 1.0, _f32)

    def row_body(j, carry):
        for k in range(LANE // 16):
            sl = pl.ds(k * 16, 16)
            si = src_v[j, sl]
            di = dst_v[j, sl]
            ils = plsc.load_gather(ilp_v, [si])
            ild = plsc.load_gather(ilp_v, [di])
            ihs = plsc.load_gather(ihp_v, [si])
            ihd = plsc.load_gather(ihp_v, [di])
            olp_v[j, sl] = (wlp_v[j, sl] + EOS) * ils * ild
            ohp_v[j, sl] = (-ALPHA) * ((whp_v[j, sl] + EOS) * ihs * ihd)
            fidx_v[j, sl] = si * NNODES + di
        pltpu.sync_copy(ones_v, adj_hbm.at[fidx_v.at[j]])
        return carry

    lax.fori_loop(0, nrows, row_body, 0)

    pltpu.sync_copy(olp_v, olp_hbm.at[pl.ds(start, TROWS)])
    pltpu.sync_copy(ohp_v, ohp_hbm.at[pl.ds(start, TROWS)])


def kernel(features, edges, eps, W1, b1, W_edge, b_edge):
    src = edges[0].astype(_i32)
    dst = edges[1].astype(_i32)
    pad = ((0, RPAD - ROWS), (0, 0))
    src2 = jnp.pad(src.reshape(ROWS, LANE), pad)
    dst2 = jnp.pad(dst.reshape(ROWS, LANE), pad)
    eps2 = jnp.pad(eps.reshape(ROWS, LANE), pad)

    q2, g2 = _tc_prep(features, W1, b1.reshape(1, HID), W_edge,
                      b_edge.reshape(1, 1), eps2)
    q = q2.reshape(NNODES)

    wlp2, whp2, dlp0, dlp1, dhp0, dhp1 = _sc_weights(q, src2, dst2, g2)
    ilp, ihp, tlp = _tc_norm(dlp0.reshape(1, NNODES), dlp1.reshape(1, NNODES),
                             dhp0.reshape(1, NNODES), dhp1.reshape(1, NNODES))

    adj_ref = jax.new_ref(jnp.zeros((NNODES * NNODES,), _f32))
    olp2, ohp2 = _sc_norm_scatter(ilp.reshape(NNODES), ihp.reshape(NNODES),
                                  src2, dst2, wlp2, whp2, adj_ref)
    adj = adj_ref[...].reshape(NNODES, NNODES)

    weights_lp = wlp2[:ROWS].reshape(NEDGES)
    weights_hp = whp2[:ROWS].reshape(NEDGES)
    w_lp_norm = jnp.concatenate([olp2[:ROWS].reshape(NEDGES), tlp.reshape(NNODES)])
    w_hp_norm = jnp.concatenate([ohp2[:ROWS].reshape(NEDGES),
                                 jnp.ones((NNODES,), _f32)])
    return (w_lp_norm, w_hp_norm, weights_lp, weights_hp, adj)
